# baseline (speedup 1.0000x reference)
import functools

import jax
import jax.numpy as jnp
from jax.experimental import pallas as pl
from jax.experimental.pallas import tpu as pltpu


def _se_block_math(xb, p_ref, *, C):
    s = jnp.sum(xb, axis=2, keepdims=True)
    m = jnp.max(xb, axis=2, keepdims=True)

    w1a = p_ref[0:C, :]
    w1m = p_ref[C:2 * C, :]
    b1c = p_ref[2 * C:3 * C, :]
    w2g = p_ref[3 * C:4 * C, :]
    w2b = p_ref[4 * C:5 * C, :]
    b2gc = p_ref[5 * C:6 * C, :]
    b2bc = p_ref[6 * C:7 * C, :]

    t = s * w1a + m * w1m + b1c
    h = jnp.sum(t, axis=1, keepdims=True)
    h = h * jnp.tanh(jax.nn.softplus(h))

    gam = jnp.sum(w2g * h + b2gc, axis=2, keepdims=True)
    bet = jnp.sum(w2b * h + b2bc, axis=2, keepdims=True)
    scale = jax.nn.sigmoid(gam)
    return scale * xb + bet


def _se_pipe(x_hbm, w1_ref, b1_ref, w2_ref, b2_ref, o_hbm,
             p_ref, x_buf, o_buf, in_sem, out_sem,
             *, inv_hw, C, hidden, bt, n_steps):
    p_ref[0:C, :] = w1_ref[:, 0:C].T * inv_hw
    p_ref[C:2 * C, :] = w1_ref[:, C:2 * C].T
    p_ref[2 * C:3 * C, :] = jnp.broadcast_to(
        b1_ref[...] * (1.0 / C), (C, hidden))
    p_ref[3 * C:4 * C, :] = w2_ref[0:C, :]
    p_ref[4 * C:5 * C, :] = w2_ref[C:2 * C, :]
    p_ref[5 * C:6 * C, :] = jnp.broadcast_to(
        b2_ref[0:C, :] * (1.0 / hidden), (C, hidden))
    p_ref[6 * C:7 * C, :] = jnp.broadcast_to(
        b2_ref[C:2 * C, :] * (1.0 / hidden), (C, hidden))
    def dma_in(slot, step):
        pltpu.make_async_copy(x_hbm.at[pl.ds(step * bt, bt)],
                              x_buf.at[slot], in_sem.at[slot]).start()

    def wait_in(slot):
        pltpu.make_async_copy(x_hbm.at[pl.ds(0, bt)],
                              x_buf.at[slot], in_sem.at[slot]).wait()

    def dma_out(slot, step):
        pltpu.make_async_copy(o_buf.at[slot],
                              o_hbm.at[pl.ds(step * bt, bt)],
                              out_sem.at[slot]).start()

    def wait_out(slot):
        pltpu.make_async_copy(o_buf.at[slot],
                              o_hbm.at[pl.ds(0, bt)],
                              out_sem.at[slot]).wait()

    dma_in(0, 0)
    if n_steps > 1:
        dma_in(1, 1)
    for k in range(n_steps):
        cur = k % 2
        wait_in(cur)
        if k >= 2:
            wait_out(cur)
        y = _se_block_math(x_buf[cur], p_ref, C=C)
        o_buf[cur] = y.astype(o_buf.dtype)
        dma_out(cur, k)
        if k + 2 < n_steps:
            dma_in(cur, k + 2)
    if n_steps >= 2:
        wait_out((n_steps - 2) % 2)
    wait_out((n_steps - 1) % 2)


def kernel(x_nchw, w1, b1, w2, b2):
    B, C, H, W = x_nchw.shape
    HW = H * W
    hidden = w1.shape[0]
    x = x_nchw.reshape(B, C, HW)
    f32 = jnp.float32

    w1f = w1.astype(f32)
    b1f = b1.astype(f32).reshape(1, hidden)
    w2f = w2.astype(f32)
    b2f = b2.astype(f32).reshape(2 * C, 1)

    per_image = C * HW * x.dtype.itemsize
    bt = 1
    for d in range(1, B + 1):
        if B % d == 0 and 4 * d * per_image <= 48 * 2**20 and B // d >= 2:
            bt = d
    n_steps = B // bt

    body = functools.partial(_se_pipe, inv_hw=1.0 / HW, C=C,
                             hidden=hidden, bt=bt, n_steps=n_steps)
    vmem = pl.BlockSpec(memory_space=pltpu.MemorySpace.VMEM)
    out = pl.pallas_call(
        body,
        out_shape=jax.ShapeDtypeStruct((B, C, HW), x.dtype),
        in_specs=[pl.BlockSpec(memory_space=pl.ANY),
                  vmem, vmem, vmem, vmem],
        out_specs=pl.BlockSpec(memory_space=pl.ANY),
        scratch_shapes=[
            pltpu.VMEM((7 * C, hidden), f32),
            pltpu.VMEM((2, bt, C, HW), f32),
            pltpu.VMEM((2, bt, C, HW), f32),
            pltpu.SemaphoreType.DMA((2,)),
            pltpu.SemaphoreType.DMA((2,)),
        ],
        compiler_params=pltpu.CompilerParams(
            vmem_limit_bytes=64 * 2**20,
        ),
    )(x, w1f, b1f, w2f, b2f)

    return out.reshape(B, C, H, W)
